# baseline (speedup 1.0000x reference)
import dataclasses
import functools

import jax
import jax.numpy as jnp
from jax import lax
from jax.experimental import pallas as pl
from jax.experimental.pallas import tpu as pltpu
from jax.experimental.pallas import tpu_sc as plsc

B = 16384
D = 32
NC = 2
NS = 16
NW = NC * NS
BPW = B // NW


def _sc_gather(users, items, user_id, item_id):
  mesh = plsc.VectorSubcoreMesh(core_axis_name="c", subcore_axis_name="s")
  cp = pltpu.CompilerParams()
  if "use_tc_tiling_on_sc" in pltpu.CompilerParams.__dataclass_fields__:
    cp = dataclasses.replace(cp, use_tc_tiling_on_sc=False)

  @functools.partial(
      pl.kernel,
      mesh=mesh,
      out_type=[
          jax.ShapeDtypeStruct((B, D), jnp.float32),
          jax.ShapeDtypeStruct((B, D), jnp.float32),
      ],
      compiler_params=cp,
      scratch_types=[
          pltpu.VMEM((BPW,), jnp.int32),
          pltpu.VMEM((BPW,), jnp.int32),
          pltpu.VMEM((BPW, D), jnp.float32),
          pltpu.VMEM((BPW, D), jnp.float32),
          pltpu.SemaphoreType.DMA,
          pltpu.SemaphoreType.DMA,
      ],
  )
  def k(users_hbm, items_hbm, uid_hbm, iid_hbm, u_out, v_out,
        uid_v, iid_v, urows_v, vrows_v, sem_u, sem_v):
    wid = lax.axis_index("s") * NC + lax.axis_index("c")
    base = wid * BPW
    pltpu.sync_copy(uid_hbm.at[pl.ds(base, BPW)], uid_v)
    pltpu.sync_copy(iid_hbm.at[pl.ds(base, BPW)], iid_v)
    cu = pltpu.async_copy(users_hbm.at[uid_v], urows_v, sem_u)
    cv = pltpu.async_copy(items_hbm.at[iid_v], vrows_v, sem_v)
    cu.wait()
    cv.wait()
    pltpu.sync_copy(urows_v, u_out.at[pl.ds(base, BPW)])
    pltpu.sync_copy(vrows_v, v_out.at[pl.ds(base, BPW)])

  return k(users, items, user_id, item_id)


def _tc_loss(u_rows, v_rows, rating):
  def body(u_ref, v_ref, r_ref, o_ref):
    w = u_ref[...] * v_ref[...]
    pred = jnp.sum(w, axis=1)
    err = r_ref[...] - pred
    o_ref[0, 0] = jnp.sum(err * err) * (1.0 / B)

  return pl.pallas_call(
      body,
      out_shape=jax.ShapeDtypeStruct((1, 1), jnp.float32),
      out_specs=pl.BlockSpec(memory_space=pltpu.SMEM),
  )(u_rows, v_rows, rating)


def kernel(user_id, item_id, rating, users, items):
  u_rows, v_rows = _sc_gather(users, items, user_id, item_id)
  return _tc_loss(u_rows, v_rows, rating)[0, 0]
